# baseline (speedup 1.0000x reference)
import functools

import jax
import jax.numpy as jnp
from jax import lax
from jax.experimental import pallas as pl
from jax.experimental.pallas import tpu as pltpu
from jax.experimental.pallas import tpu_sc as plsc

_EMB = 64
_LANES = 16
_SEGS = _EMB // _LANES
_NC, _NS = 2, 16
_NW = _NC * _NS
_TILE = 128
_NTILES = 244
_EXTRA0 = _NW * _NTILES
_TAIL0 = 999936
_NBKT = 246
_BCAP = 16
_LCAP = 1024
_ICH = 4096


def _phase1(center_words, context_words, in_t, out_t, tail_in, tail_out):
    B = center_words.shape[0]
    mesh = plsc.VectorSubcoreMesh(core_axis_name="c", subcore_axis_name="s")

    @functools.partial(
        pl.kernel,
        mesh=mesh,
        compiler_params=pltpu.CompilerParams(needs_layout_passes=False,
                                             use_tc_tiling_on_sc=True),
        out_type=(jax.ShapeDtypeStruct((B, _EMB), jnp.float32),
                  jax.ShapeDtypeStruct((B, _EMB), jnp.float32)),
        scratch_types=[
            pltpu.VMEM((_ICH,), jnp.int32),
            pltpu.VMEM((4, _EMB, _TILE), jnp.float32),
            pltpu.VMEM((4, _EMB, _TILE), jnp.float32),
            pltpu.VMEM((_EMB, _TILE), jnp.float32),
            pltpu.VMEM((_EMB, _TILE), jnp.float32),
            pltpu.VMEM((_LCAP,), jnp.int32),
            pltpu.VMEM((_LCAP,), jnp.int32),
            pltpu.VMEM((_NBKT * _BCAP + 16,), jnp.int32),
            pltpu.VMEM((_NBKT * _BCAP + 16,), jnp.int32),
            pltpu.VMEM((_NBKT * _BCAP + 16,), jnp.int32),
            pltpu.VMEM((_NBKT * _BCAP + 16,), jnp.int32),
            pltpu.VMEM((256,), jnp.int32),
            pltpu.VMEM((256,), jnp.int32),
            pltpu.VMEM((16, _EMB), jnp.float32),
            pltpu.SemaphoreType.DMA,
            pltpu.SemaphoreType.DMA,
            pltpu.SemaphoreType.DMA,
            pltpu.SemaphoreType.DMA,
            pltpu.SemaphoreType.DMA,
        ],
    )
    def p1(center_hbm, context_hbm, in_hbm, out_hbm, tin_hbm, tout_hbm,
           crows_hbm, xrows_hbm,
           ibuf, wbin, wbout, tbin, tbout, mlidx, mlpos,
           cbcol, cbpos, xbcol, xbpos, ccnt, xcnt, stg,
           sem0, sem1, sem2, sem3, semo):
        wid = lax.axis_index("s") * _NC + lax.axis_index("c")
        lane = lax.iota(jnp.int32, _LANES)

        tile0 = _NTILES * wid
        lo = tile0 * _TILE
        hi = lo + _NTILES * _TILE
        has_extra = wid < 4
        elo = (_EXTRA0 + wid) * _TILE
        is_last = wid == (_NW - 1)
        lo2 = jnp.where(has_extra, elo, jnp.where(is_last, _TAIL0, -1))
        hi2 = jnp.where(has_extra, elo + _TILE,
                        jnp.where(is_last, 1 << 30, -1))

        sems = (sem0, sem1, sem2, sem3)

        def issue(w, slot):
            s = pl.ds((tile0 + w) * _TILE, _TILE)
            pltpu.async_copy(in_hbm.at[:, s], wbin.at[slot], sems[slot])
            pltpu.async_copy(out_hbm.at[:, s], wbout.at[slot], sems[slot])

        def drain(slot):
            pltpu.make_async_copy(in_hbm.at[:, pl.ds(0, _TILE)],
                                  wbin.at[slot], sems[slot]).wait()
            pltpu.make_async_copy(out_hbm.at[:, pl.ds(0, _TILE)],
                                  wbout.at[slot], sems[slot]).wait()

        for p in range(4):
            issue(p, p)

        pltpu.sync_copy(tin_hbm, tbin)
        pltpu.sync_copy(tout_hbm, tbout)

        def zero(j, z):
            ccnt[pl.ds(j * 16, 16)] = jnp.zeros((16,), jnp.int32)
            xcnt[pl.ds(j * 16, 16)] = jnp.zeros((16,), jnp.int32)
            return z

        lax.fori_loop(0, 16, zero, 0)

        def select(src_hbm):
            cnt = 0
            for ch in range(B // _ICH):
                pltpu.sync_copy(src_hbm.at[pl.ds(ch * _ICH, _ICH)], ibuf)

                def body(t, cnt):
                    v = ibuf[pl.ds(t * 16, 16)]
                    m = jnp.logical_or(
                        jnp.logical_and(v >= lo, v < hi),
                        jnp.logical_and(v >= lo2, v < hi2))
                    plsc.store_compressed(mlidx.at[pl.ds(cnt, 16)], v, mask=m)
                    plsc.store_compressed(mlpos.at[pl.ds(cnt, 16)],
                                          ch * _ICH + t * 16 + lane, mask=m)
                    n = plsc.all_reduce_population_count(m)
                    return cnt + n[0]

                cnt = lax.fori_loop(0, _ICH // 16, body, cnt, unroll=4)
            return cnt

        lane0 = lane == 0

        def bucketize(cnt, bcol, bpos, bcnt):
            def body(j, z):
                idx = mlidx[pl.ds(j, 16)][0]
                pos = mlpos[pl.ds(j, 16)][0]
                t = jnp.where(idx >= _TAIL0, _NBKT - 1,
                              jnp.where(idx >= elo, _NBKT - 2,
                                        (idx >> 7) - tile0))
                col = jnp.where(idx >= _TAIL0, idx - _TAIL0,
                                jnp.bitwise_and(idx, _TILE - 1))
                tv = jax.lax.broadcast(t, (16,))
                k = plsc.load_gather(bcnt, [tv])[0]
                sv16 = jax.lax.broadcast(t * _BCAP + k, (16,))
                plsc.store_scatter(bcol, [sv16],
                                   jax.lax.broadcast(col, (16,)), mask=lane0)
                plsc.store_scatter(bpos, [sv16],
                                   jax.lax.broadcast(pos, (16,)), mask=lane0)
                plsc.store_scatter(bcnt, [tv],
                                   jax.lax.broadcast(k + 1, (16,)), mask=lane0)
                return z

            lax.fori_loop(0, cnt, body, 0)

        ncm = select(center_hbm)
        bucketize(ncm, cbcol, cbpos, ccnt)
        nxm = select(context_hbm)
        bucketize(nxm, xbcol, xbpos, xcnt)

        def extract(w, src, bcol, bpos, bcnt, rows_hbm, ne0):
            k = plsc.load_gather(bcnt, [jax.lax.broadcast(w, (16,))])[0]

            def ent(j, ne):
                col = bcol[pl.ds(w * _BCAP + j, 16)][0]
                pos = bpos[pl.ds(w * _BCAP + j, 16)][0]
                s = jnp.bitwise_and(ne, 15)

                @pl.when(ne >= 16)
                def _():
                    pltpu.make_async_copy(stg.at[0], rows_hbm.at[0],
                                          semo).wait()

                cv = jax.lax.broadcast(col, (16,))
                for seg in range(_SEGS):
                    g = plsc.load_gather(src, [seg * 16 + lane, cv])
                    stg[s, pl.ds(seg * 16, 16)] = g
                pltpu.async_copy(stg.at[s], rows_hbm.at[pos], semo)
                return ne + 1

            return lax.fori_loop(0, k, ent, ne0)

        def step(u, ne):
            for p in range(4):
                w = 4 * u + p
                drain(p)
                ne = extract(w, wbin.at[p], cbcol, cbpos, ccnt, crows_hbm, ne)
                ne = extract(w, wbout.at[p], xbcol, xbpos, xcnt, xrows_hbm, ne)

                @pl.when(w + 4 < _NTILES)
                def _():
                    issue(w + 4, p)

            return ne

        ne = lax.fori_loop(0, _NTILES // 4, step, 0)

        def extra_win(n):
            s = pl.ds((_EXTRA0 + wid) * _TILE, _TILE)
            pltpu.async_copy(in_hbm.at[:, s], wbin.at[0], sems[0])
            pltpu.async_copy(out_hbm.at[:, s], wbout.at[0], sems[0])
            drain(0)
            n = extract(_NBKT - 2, wbin.at[0], cbcol, cbpos, ccnt,
                        crows_hbm, n)
            n = extract(_NBKT - 2, wbout.at[0], xbcol, xbpos, xcnt,
                        xrows_hbm, n)
            return n

        ne = lax.cond(has_extra, extra_win, lambda n: n, ne)

        ne = extract(_NBKT - 1, tbin, cbcol, cbpos, ccnt, crows_hbm, ne)
        ne = extract(_NBKT - 1, tbout, xbcol, xbpos, xcnt, xrows_hbm, ne)

        def fin(j, z):
            pltpu.make_async_copy(stg.at[0], crows_hbm.at[0], semo).wait()
            return z

        lax.fori_loop(0, jnp.minimum(ne, 16), fin, 0)

    return p1


def _phase2(B):
    b_per_w = B // _NW
    CH = 128
    n_chunks = b_per_w // CH
    mesh = plsc.VectorSubcoreMesh(core_axis_name="c", subcore_axis_name="s")

    @functools.partial(
        pl.kernel,
        mesh=mesh,
        compiler_params=pltpu.CompilerParams(needs_layout_passes=False,
                                             use_tc_tiling_on_sc=True),
        out_type=jax.ShapeDtypeStruct((B,), jnp.float32),
        scratch_types=[
            pltpu.VMEM((2, CH, _EMB), jnp.float32),
            pltpu.VMEM((2, CH, _EMB), jnp.float32),
            pltpu.VMEM((b_per_w,), jnp.float32),
            pltpu.SemaphoreType.DMA,
            pltpu.SemaphoreType.DMA,
        ],
    )
    def p2(crows_hbm, xrows_hbm, scores_hbm, cbuf, xbuf, sv, sem0, sem1):
        wid = lax.axis_index("s") * _NC + lax.axis_index("c")
        base = wid * b_per_w
        sems = (sem0, sem1)
        lane = lax.iota(jnp.int32, _LANES)

        def issue(c, slot):
            s = pl.ds(base + c * CH, CH)
            pltpu.async_copy(crows_hbm.at[s], cbuf.at[slot], sems[slot])
            pltpu.async_copy(xrows_hbm.at[s], xbuf.at[slot], sems[slot])

        def drain(slot):
            pltpu.make_async_copy(crows_hbm.at[pl.ds(0, CH)],
                                  cbuf.at[slot], sems[slot]).wait()
            pltpu.make_async_copy(xrows_hbm.at[pl.ds(0, CH)],
                                  xbuf.at[slot], sems[slot]).wait()

        def compute(c, slot):
            for g in range(CH // _LANES):
                res = jnp.zeros((_LANES,), jnp.float32)
                for i in range(_LANES):
                    li = g * _LANES + i
                    acc = (cbuf[slot, li, pl.ds(0, _LANES)]
                           * xbuf[slot, li, pl.ds(0, _LANES)])
                    for s in range(1, _SEGS):
                        acc = acc + (
                            cbuf[slot, li, pl.ds(s * _LANES, _LANES)]
                            * xbuf[slot, li, pl.ds(s * _LANES, _LANES)])
                    res = jnp.where(lane == i, jnp.sum(acc), res)
                sv[pl.ds(c * CH + g * _LANES, _LANES)] = res

        issue(0, 0)
        issue(1, 1)

        def step(t, carry):
            drain(0)
            compute(2 * t, 0)

            @pl.when(t < n_chunks // 2 - 1)
            def _():
                issue(2 * t + 2, 0)

            drain(1)
            compute(2 * t + 1, 1)

            @pl.when(t < n_chunks // 2 - 1)
            def _():
                issue(2 * t + 3, 1)

            return carry

        lax.fori_loop(0, n_chunks // 2, step, 0)
        pltpu.sync_copy(sv, scores_hbm.at[pl.ds(base, b_per_w)])

    return p2


def kernel(center_words, context_words, in_emb, out_emb):
    B = center_words.shape[0]
    in_t = in_emb.T
    out_t = out_emb.T
    tail_in = jnp.pad(in_emb[_TAIL0:].T, ((0, 0), (0, 64)))
    tail_out = jnp.pad(out_emb[_TAIL0:].T, ((0, 0), (0, 64)))
    crows, xrows = _phase1(center_words, context_words, in_t, out_t,
                           tail_in, tail_out)(
        center_words, context_words, in_t, out_t, tail_in, tail_out)
    return _phase2(B)(crows, xrows)


# speedup vs baseline: 1.0180x; 1.0180x over previous
import functools

import jax
import jax.numpy as jnp
from jax import lax
from jax.experimental import pallas as pl
from jax.experimental.pallas import tpu as pltpu
from jax.experimental.pallas import tpu_sc as plsc

_EMB = 64
_LANES = 16
_SEGS = _EMB // _LANES
_NC, _NS = 2, 16
_NW = _NC * _NS
_TILE = 128
_NTILES = 244
_EXTRA0 = _NW * _NTILES
_TAIL0 = 999936
_NBKT = 246
_BCAP = 16
_LCAP = 1024
_ICH = 4096


def _phase1(center_words, context_words, in_t, out_t, tail_in, tail_out):
    B = center_words.shape[0]
    mesh = plsc.VectorSubcoreMesh(core_axis_name="c", subcore_axis_name="s")

    @functools.partial(
        pl.kernel,
        mesh=mesh,
        compiler_params=pltpu.CompilerParams(needs_layout_passes=False,
                                             use_tc_tiling_on_sc=True),
        out_type=(jax.ShapeDtypeStruct((B, _EMB), jnp.float32),
                  jax.ShapeDtypeStruct((B, _EMB), jnp.float32)),
        scratch_types=[
            pltpu.VMEM((B,), jnp.int32),
            pltpu.VMEM((B,), jnp.int32),
            pltpu.VMEM((4, _EMB, _TILE), jnp.float32),
            pltpu.VMEM((4, _EMB, _TILE), jnp.float32),
            pltpu.VMEM((_LCAP,), jnp.int32),
            pltpu.VMEM((_LCAP,), jnp.int32),
            pltpu.VMEM((_NBKT * _BCAP + 16,), jnp.int32),
            pltpu.VMEM((_NBKT * _BCAP + 16,), jnp.int32),
            pltpu.VMEM((_NBKT * _BCAP + 16,), jnp.int32),
            pltpu.VMEM((_NBKT * _BCAP + 16,), jnp.int32),
            pltpu.VMEM((256,), jnp.int32),
            pltpu.VMEM((256,), jnp.int32),
            pltpu.VMEM((16, _EMB), jnp.float32),
            pltpu.SemaphoreType.DMA,
            pltpu.SemaphoreType.DMA,
            pltpu.SemaphoreType.DMA,
            pltpu.SemaphoreType.DMA,
            pltpu.SemaphoreType.DMA,
        ],
    )
    def p1(center_hbm, context_hbm, in_hbm, out_hbm, tin_hbm, tout_hbm,
           crows_hbm, xrows_hbm,
           cidx, xidx, wbin, wbout, mlidx, mlpos,
           cbcol, cbpos, xbcol, xbpos, ccnt, xcnt, stg,
           sem0, sem1, sem2, sem3, semo):
        wid = lax.axis_index("s") * _NC + lax.axis_index("c")
        lane = lax.iota(jnp.int32, _LANES)

        tile0 = _NTILES * wid
        lo = tile0 * _TILE
        hi = lo + _NTILES * _TILE
        has_extra = wid < 4
        elo = (_EXTRA0 + wid) * _TILE
        is_last = wid == (_NW - 1)
        lo2 = jnp.where(has_extra, elo, jnp.where(is_last, _TAIL0, -1))
        hi2 = jnp.where(has_extra, elo + _TILE,
                        jnp.where(is_last, 1 << 30, -1))

        sems = (sem0, sem1, sem2, sem3)

        def issue(w, slot):
            s = pl.ds((tile0 + w) * _TILE, _TILE)
            pltpu.async_copy(in_hbm.at[:, s], wbin.at[slot], sems[slot])
            pltpu.async_copy(out_hbm.at[:, s], wbout.at[slot], sems[slot])

        def drain(slot):
            pltpu.make_async_copy(in_hbm.at[:, pl.ds(0, _TILE)],
                                  wbin.at[slot], sems[slot]).wait()
            pltpu.make_async_copy(out_hbm.at[:, pl.ds(0, _TILE)],
                                  wbout.at[slot], sems[slot]).wait()

        for p in range(4):
            issue(p, p)

        pltpu.sync_copy(center_hbm, cidx)
        pltpu.sync_copy(context_hbm, xidx)

        def zero(j, z):
            ccnt[pl.ds(j * 16, 16)] = jnp.zeros((16,), jnp.int32)
            xcnt[pl.ds(j * 16, 16)] = jnp.zeros((16,), jnp.int32)
            return z

        lax.fori_loop(0, 16, zero, 0)

        def select(ibuf):
            def body(t, cnt):
                v = ibuf[pl.ds(t * 16, 16)]
                m = jnp.logical_or(
                    jnp.logical_and(v >= lo, v < hi),
                    jnp.logical_and(v >= lo2, v < hi2))
                plsc.store_compressed(mlidx.at[pl.ds(cnt, 16)], v, mask=m)
                plsc.store_compressed(mlpos.at[pl.ds(cnt, 16)],
                                      t * 16 + lane, mask=m)
                n = plsc.all_reduce_population_count(m)
                return cnt + n[0]

            return lax.fori_loop(0, B // 16, body, 0, unroll=4)

        lane0 = lane == 0

        def bucketize(cnt, bcol, bpos, bcnt):
            def body(j, z):
                idx = mlidx[pl.ds(j, 16)][0]
                pos = mlpos[pl.ds(j, 16)][0]
                t = jnp.where(idx >= _TAIL0, _NBKT - 1,
                              jnp.where(idx >= elo, _NBKT - 2,
                                        (idx >> 7) - tile0))
                col = jnp.where(idx >= _TAIL0, idx - _TAIL0,
                                jnp.bitwise_and(idx, _TILE - 1))
                tv = jax.lax.broadcast(t, (16,))
                k = plsc.load_gather(bcnt, [tv])[0]
                sv16 = jax.lax.broadcast(t * _BCAP + k, (16,))
                plsc.store_scatter(bcol, [sv16],
                                   jax.lax.broadcast(col, (16,)), mask=lane0)
                plsc.store_scatter(bpos, [sv16],
                                   jax.lax.broadcast(pos, (16,)), mask=lane0)
                plsc.store_scatter(bcnt, [tv],
                                   jax.lax.broadcast(k + 1, (16,)), mask=lane0)
                return z

            lax.fori_loop(0, cnt, body, 0)

        ncm = select(cidx)
        bucketize(ncm, cbcol, cbpos, ccnt)
        nxm = select(xidx)
        bucketize(nxm, xbcol, xbpos, xcnt)

        def extract(w, src, bcol, bpos, bcnt, rows_hbm, ne0):
            k = plsc.load_gather(bcnt, [jax.lax.broadcast(w, (16,))])[0]

            def ent(j, ne):
                col = bcol[pl.ds(w * _BCAP + j, 16)][0]
                pos = bpos[pl.ds(w * _BCAP + j, 16)][0]
                s = jnp.bitwise_and(ne, 15)

                @pl.when(ne >= 16)
                def _():
                    pltpu.make_async_copy(stg.at[0], rows_hbm.at[0],
                                          semo).wait()

                cv = jax.lax.broadcast(col, (16,))
                for seg in range(_SEGS):
                    g = plsc.load_gather(src, [seg * 16 + lane, cv])
                    stg[s, pl.ds(seg * 16, 16)] = g
                pltpu.async_copy(stg.at[s], rows_hbm.at[pos], semo)
                return ne + 1

            return lax.fori_loop(0, k, ent, ne0)

        def step(u, ne):
            for p in range(4):
                w = 4 * u + p
                drain(p)
                ne = extract(w, wbin.at[p], cbcol, cbpos, ccnt, crows_hbm, ne)
                ne = extract(w, wbout.at[p], xbcol, xbpos, xcnt, xrows_hbm, ne)

                @pl.when(w + 4 < _NTILES)
                def _():
                    issue(w + 4, p)

            return ne

        ne = lax.fori_loop(0, _NTILES // 4, step, 0)

        def extra_win(n):
            s = pl.ds((_EXTRA0 + wid) * _TILE, _TILE)
            pltpu.async_copy(in_hbm.at[:, s], wbin.at[0], sems[0])
            pltpu.async_copy(out_hbm.at[:, s], wbout.at[0], sems[0])
            drain(0)
            n = extract(_NBKT - 2, wbin.at[0], cbcol, cbpos, ccnt,
                        crows_hbm, n)
            n = extract(_NBKT - 2, wbout.at[0], xbcol, xbpos, xcnt,
                        xrows_hbm, n)
            return n

        ne = lax.cond(has_extra, extra_win, lambda n: n, ne)

        @pl.when(is_last)
        def _():
            pltpu.sync_copy(tin_hbm, wbin.at[1])
            pltpu.sync_copy(tout_hbm, wbout.at[1])

        ne = extract(_NBKT - 1, wbin.at[1], cbcol, cbpos, ccnt,
                     crows_hbm, ne)
        ne = extract(_NBKT - 1, wbout.at[1], xbcol, xbpos, xcnt,
                     xrows_hbm, ne)

        def fin(j, z):
            pltpu.make_async_copy(stg.at[0], crows_hbm.at[0], semo).wait()
            return z

        lax.fori_loop(0, jnp.minimum(ne, 16), fin, 0)

    return p1


def _phase2(B):
    b_per_w = B // _NW
    CH = 128
    n_chunks = b_per_w // CH
    mesh = plsc.VectorSubcoreMesh(core_axis_name="c", subcore_axis_name="s")

    @functools.partial(
        pl.kernel,
        mesh=mesh,
        compiler_params=pltpu.CompilerParams(needs_layout_passes=False,
                                             use_tc_tiling_on_sc=True),
        out_type=jax.ShapeDtypeStruct((B,), jnp.float32),
        scratch_types=[
            pltpu.VMEM((2, CH, _EMB), jnp.float32),
            pltpu.VMEM((2, CH, _EMB), jnp.float32),
            pltpu.VMEM((b_per_w,), jnp.float32),
            pltpu.SemaphoreType.DMA,
            pltpu.SemaphoreType.DMA,
        ],
    )
    def p2(crows_hbm, xrows_hbm, scores_hbm, cbuf, xbuf, sv, sem0, sem1):
        wid = lax.axis_index("s") * _NC + lax.axis_index("c")
        base = wid * b_per_w
        sems = (sem0, sem1)
        lane = lax.iota(jnp.int32, _LANES)

        def issue(c, slot):
            s = pl.ds(base + c * CH, CH)
            pltpu.async_copy(crows_hbm.at[s], cbuf.at[slot], sems[slot])
            pltpu.async_copy(xrows_hbm.at[s], xbuf.at[slot], sems[slot])

        def drain(slot):
            pltpu.make_async_copy(crows_hbm.at[pl.ds(0, CH)],
                                  cbuf.at[slot], sems[slot]).wait()
            pltpu.make_async_copy(xrows_hbm.at[pl.ds(0, CH)],
                                  xbuf.at[slot], sems[slot]).wait()

        def compute(c, slot):
            for g in range(CH // _LANES):
                res = jnp.zeros((_LANES,), jnp.float32)
                for i in range(_LANES):
                    li = g * _LANES + i
                    acc = (cbuf[slot, li, pl.ds(0, _LANES)]
                           * xbuf[slot, li, pl.ds(0, _LANES)])
                    for s in range(1, _SEGS):
                        acc = acc + (
                            cbuf[slot, li, pl.ds(s * _LANES, _LANES)]
                            * xbuf[slot, li, pl.ds(s * _LANES, _LANES)])
                    res = jnp.where(lane == i, jnp.sum(acc), res)
                sv[pl.ds(c * CH + g * _LANES, _LANES)] = res

        issue(0, 0)
        issue(1, 1)

        def step(t, carry):
            drain(0)
            compute(2 * t, 0)

            @pl.when(t < n_chunks // 2 - 1)
            def _():
                issue(2 * t + 2, 0)

            drain(1)
            compute(2 * t + 1, 1)

            @pl.when(t < n_chunks // 2 - 1)
            def _():
                issue(2 * t + 3, 1)

            return carry

        lax.fori_loop(0, n_chunks // 2, step, 0)
        pltpu.sync_copy(sv, scores_hbm.at[pl.ds(base, b_per_w)])

    return p2


def kernel(center_words, context_words, in_emb, out_emb):
    B = center_words.shape[0]
    in_t = in_emb.T
    out_t = out_emb.T
    tail_in = jnp.pad(in_emb[_TAIL0:].T, ((0, 0), (0, 64)))
    tail_out = jnp.pad(out_emb[_TAIL0:].T, ((0, 0), (0, 64)))
    crows, xrows = _phase1(center_words, context_words, in_t, out_t,
                           tail_in, tail_out)(
        center_words, context_words, in_t, out_t, tail_in, tail_out)
    return _phase2(B)(crows, xrows)


# speedup vs baseline: 1.0571x; 1.0385x over previous
import functools

import jax
import jax.numpy as jnp
from jax import lax
from jax.experimental import pallas as pl
from jax.experimental.pallas import tpu as pltpu
from jax.experimental.pallas import tpu_sc as plsc

_EMB = 64
_LANES = 16
_SEGS = _EMB // _LANES
_NC, _NS = 2, 16
_NW = _NC * _NS
_TILE = 128
_NTILES = 244
_EXTRA0 = _NW * _NTILES
_TAIL0 = 999936
_NBKT = 246
_BCAP = 16
_LCAP = 1024
_ICH = 4096


def _phase1(center_words, context_words, in_t, out_t, tail_in, tail_out):
    B = center_words.shape[0]
    mesh = plsc.VectorSubcoreMesh(core_axis_name="c", subcore_axis_name="s")

    @functools.partial(
        pl.kernel,
        mesh=mesh,
        compiler_params=pltpu.CompilerParams(needs_layout_passes=False,
                                             use_tc_tiling_on_sc=True),
        out_type=(jax.ShapeDtypeStruct((B, _EMB), jnp.float32),
                  jax.ShapeDtypeStruct((B, _EMB), jnp.float32)),
        scratch_types=[
            pltpu.VMEM((B,), jnp.int32),
            pltpu.VMEM((5, _EMB, _TILE), jnp.float32),
            pltpu.VMEM((5, _EMB, _TILE), jnp.float32),
            pltpu.VMEM((_LCAP,), jnp.int32),
            pltpu.VMEM((_LCAP,), jnp.int32),
            pltpu.VMEM((_NBKT * _BCAP + 16,), jnp.int32),
            pltpu.VMEM((_NBKT * _BCAP + 16,), jnp.int32),
            pltpu.VMEM((_NBKT * _BCAP + 16,), jnp.int32),
            pltpu.VMEM((_NBKT * _BCAP + 16,), jnp.int32),
            pltpu.VMEM((256,), jnp.int32),
            pltpu.VMEM((256,), jnp.int32),
            pltpu.VMEM((16, _EMB), jnp.float32),
            pltpu.SemaphoreType.DMA,
            pltpu.SemaphoreType.DMA,
            pltpu.SemaphoreType.DMA,
            pltpu.SemaphoreType.DMA,
            pltpu.SemaphoreType.DMA,
            pltpu.SemaphoreType.DMA,
        ],
    )
    def p1(center_hbm, context_hbm, in_hbm, out_hbm, tin_hbm, tout_hbm,
           crows_hbm, xrows_hbm,
           idxbuf, wbin, wbout, mlidx, mlpos,
           cbcol, cbpos, xbcol, xbpos, ccnt, xcnt, stg,
           sem0, sem1, sem2, sem3, sem4, semo):
        wid = lax.axis_index("s") * _NC + lax.axis_index("c")
        lane = lax.iota(jnp.int32, _LANES)

        tile0 = _NTILES * wid
        lo = tile0 * _TILE
        hi = lo + _NTILES * _TILE
        has_extra = wid < 4
        elo = (_EXTRA0 + wid) * _TILE
        is_last = wid == (_NW - 1)
        lo2 = jnp.where(has_extra, elo, jnp.where(is_last, _TAIL0, -1))
        hi2 = jnp.where(has_extra, elo + _TILE,
                        jnp.where(is_last, 1 << 30, -1))

        sems = (sem0, sem1, sem2, sem3, sem4)

        def issue(w, slot):
            s = pl.ds((tile0 + w) * _TILE, _TILE)
            pltpu.async_copy(in_hbm.at[:, s], wbin.at[slot], sems[slot])
            pltpu.async_copy(out_hbm.at[:, s], wbout.at[slot], sems[slot])

        def drain(slot):
            pltpu.make_async_copy(in_hbm.at[:, pl.ds(0, _TILE)],
                                  wbin.at[slot], sems[slot]).wait()
            pltpu.make_async_copy(out_hbm.at[:, pl.ds(0, _TILE)],
                                  wbout.at[slot], sems[slot]).wait()

        for p in range(5):
            issue(p, p)

        def zero(j, z):
            ccnt[pl.ds(j * 16, 16)] = jnp.zeros((16,), jnp.int32)
            xcnt[pl.ds(j * 16, 16)] = jnp.zeros((16,), jnp.int32)
            return z

        lax.fori_loop(0, 16, zero, 0)

        def select(ibuf):
            def body(t, cnt):
                v = ibuf[pl.ds(t * 16, 16)]
                m = jnp.logical_or(
                    jnp.logical_and(v >= lo, v < hi),
                    jnp.logical_and(v >= lo2, v < hi2))
                plsc.store_compressed(mlidx.at[pl.ds(cnt, 16)], v, mask=m)
                plsc.store_compressed(mlpos.at[pl.ds(cnt, 16)],
                                      t * 16 + lane, mask=m)
                n = plsc.all_reduce_population_count(m)
                return cnt + n[0]

            return lax.fori_loop(0, B // 16, body, 0, unroll=4)

        lane0 = lane == 0

        def bucketize(cnt, bcol, bpos, bcnt):
            def body(j, z):
                idx = mlidx[pl.ds(j, 16)][0]
                pos = mlpos[pl.ds(j, 16)][0]
                t = jnp.where(idx >= _TAIL0, _NBKT - 1,
                              jnp.where(idx >= elo, _NBKT - 2,
                                        (idx >> 7) - tile0))
                col = jnp.where(idx >= _TAIL0, idx - _TAIL0,
                                jnp.bitwise_and(idx, _TILE - 1))
                tv = jax.lax.broadcast(t, (16,))
                k = plsc.load_gather(bcnt, [tv])[0]
                sv16 = jax.lax.broadcast(t * _BCAP + k, (16,))
                plsc.store_scatter(bcol, [sv16],
                                   jax.lax.broadcast(col, (16,)), mask=lane0)
                plsc.store_scatter(bpos, [sv16],
                                   jax.lax.broadcast(pos, (16,)), mask=lane0)
                plsc.store_scatter(bcnt, [tv],
                                   jax.lax.broadcast(k + 1, (16,)), mask=lane0)
                return z

            lax.fori_loop(0, cnt, body, 0)

        pltpu.sync_copy(center_hbm, idxbuf)
        ncm = select(idxbuf)
        bucketize(ncm, cbcol, cbpos, ccnt)
        pltpu.sync_copy(context_hbm, idxbuf)
        nxm = select(idxbuf)
        bucketize(nxm, xbcol, xbpos, xcnt)

        def extract(w, src, bcol, bpos, bcnt, rows_hbm, ne0):
            k = plsc.load_gather(bcnt, [jax.lax.broadcast(w, (16,))])[0]

            def ent(j, ne):
                col = bcol[pl.ds(w * _BCAP + j, 16)][0]
                pos = bpos[pl.ds(w * _BCAP + j, 16)][0]
                s = jnp.bitwise_and(ne, 15)

                @pl.when(ne >= 16)
                def _():
                    pltpu.make_async_copy(stg.at[0], rows_hbm.at[0],
                                          semo).wait()

                cv = jax.lax.broadcast(col, (16,))
                for seg in range(_SEGS):
                    g = plsc.load_gather(src, [seg * 16 + lane, cv])
                    stg[s, pl.ds(seg * 16, 16)] = g
                pltpu.async_copy(stg.at[s], rows_hbm.at[pos], semo)
                return ne + 1

            return lax.fori_loop(0, k, ent, ne0)

        def step(u, ne):
            for p in range(5):
                w = 5 * u + p
                drain(p)
                ne = extract(w, wbin.at[p], cbcol, cbpos, ccnt, crows_hbm, ne)
                ne = extract(w, wbout.at[p], xbcol, xbpos, xcnt, xrows_hbm, ne)

                @pl.when(w + 5 < _NTILES)
                def _():
                    issue(w + 5, p)

            return ne

        ne = lax.fori_loop(0, _NTILES // 5, step, 0)

        for p in range(4):
            w = (_NTILES // 5) * 5 + p
            drain(p)
            ne = extract(w, wbin.at[p], cbcol, cbpos, ccnt, crows_hbm, ne)
            ne = extract(w, wbout.at[p], xbcol, xbpos, xcnt, xrows_hbm, ne)

        def extra_win(n):
            s = pl.ds((_EXTRA0 + wid) * _TILE, _TILE)
            pltpu.async_copy(in_hbm.at[:, s], wbin.at[0], sems[0])
            pltpu.async_copy(out_hbm.at[:, s], wbout.at[0], sems[0])
            drain(0)
            n = extract(_NBKT - 2, wbin.at[0], cbcol, cbpos, ccnt,
                        crows_hbm, n)
            n = extract(_NBKT - 2, wbout.at[0], xbcol, xbpos, xcnt,
                        xrows_hbm, n)
            return n

        ne = lax.cond(has_extra, extra_win, lambda n: n, ne)

        @pl.when(is_last)
        def _():
            pltpu.sync_copy(tin_hbm, wbin.at[1])
            pltpu.sync_copy(tout_hbm, wbout.at[1])

        ne = extract(_NBKT - 1, wbin.at[1], cbcol, cbpos, ccnt,
                     crows_hbm, ne)
        ne = extract(_NBKT - 1, wbout.at[1], xbcol, xbpos, xcnt,
                     xrows_hbm, ne)

        def fin(j, z):
            pltpu.make_async_copy(stg.at[0], crows_hbm.at[0], semo).wait()
            return z

        lax.fori_loop(0, jnp.minimum(ne, 16), fin, 0)

    return p1


def _phase2(B):
    b_per_w = B // _NW
    CH = 128
    n_chunks = b_per_w // CH
    mesh = plsc.VectorSubcoreMesh(core_axis_name="c", subcore_axis_name="s")

    @functools.partial(
        pl.kernel,
        mesh=mesh,
        compiler_params=pltpu.CompilerParams(needs_layout_passes=False,
                                             use_tc_tiling_on_sc=True),
        out_type=jax.ShapeDtypeStruct((B,), jnp.float32),
        scratch_types=[
            pltpu.VMEM((2, CH, _EMB), jnp.float32),
            pltpu.VMEM((2, CH, _EMB), jnp.float32),
            pltpu.VMEM((b_per_w,), jnp.float32),
            pltpu.SemaphoreType.DMA,
            pltpu.SemaphoreType.DMA,
        ],
    )
    def p2(crows_hbm, xrows_hbm, scores_hbm, cbuf, xbuf, sv, sem0, sem1):
        wid = lax.axis_index("s") * _NC + lax.axis_index("c")
        base = wid * b_per_w
        sems = (sem0, sem1)
        lane = lax.iota(jnp.int32, _LANES)

        def issue(c, slot):
            s = pl.ds(base + c * CH, CH)
            pltpu.async_copy(crows_hbm.at[s], cbuf.at[slot], sems[slot])
            pltpu.async_copy(xrows_hbm.at[s], xbuf.at[slot], sems[slot])

        def drain(slot):
            pltpu.make_async_copy(crows_hbm.at[pl.ds(0, CH)],
                                  cbuf.at[slot], sems[slot]).wait()
            pltpu.make_async_copy(xrows_hbm.at[pl.ds(0, CH)],
                                  xbuf.at[slot], sems[slot]).wait()

        def compute(c, slot):
            for g in range(CH // _LANES):
                res = jnp.zeros((_LANES,), jnp.float32)
                for i in range(_LANES):
                    li = g * _LANES + i
                    acc = (cbuf[slot, li, pl.ds(0, _LANES)]
                           * xbuf[slot, li, pl.ds(0, _LANES)])
                    for s in range(1, _SEGS):
                        acc = acc + (
                            cbuf[slot, li, pl.ds(s * _LANES, _LANES)]
                            * xbuf[slot, li, pl.ds(s * _LANES, _LANES)])
                    res = jnp.where(lane == i, jnp.sum(acc), res)
                sv[pl.ds(c * CH + g * _LANES, _LANES)] = res

        issue(0, 0)
        issue(1, 1)

        def step(t, carry):
            drain(0)
            compute(2 * t, 0)

            @pl.when(t < n_chunks // 2 - 1)
            def _():
                issue(2 * t + 2, 0)

            drain(1)
            compute(2 * t + 1, 1)

            @pl.when(t < n_chunks // 2 - 1)
            def _():
                issue(2 * t + 3, 1)

            return carry

        lax.fori_loop(0, n_chunks // 2, step, 0)
        pltpu.sync_copy(sv, scores_hbm.at[pl.ds(base, b_per_w)])

    return p2


def kernel(center_words, context_words, in_emb, out_emb):
    B = center_words.shape[0]
    in_t = in_emb.T
    out_t = out_emb.T
    tail_in = jnp.pad(in_emb[_TAIL0:].T, ((0, 0), (0, 64)))
    tail_out = jnp.pad(out_emb[_TAIL0:].T, ((0, 0), (0, 64)))
    crows, xrows = _phase1(center_words, context_words, in_t, out_t,
                           tail_in, tail_out)(
        center_words, context_words, in_t, out_t, tail_in, tail_out)
    return _phase2(B)(crows, xrows)


# speedup vs baseline: 1.0575x; 1.0004x over previous
import functools

import jax
import jax.numpy as jnp
from jax import lax
from jax.experimental import pallas as pl
from jax.experimental.pallas import tpu as pltpu
from jax.experimental.pallas import tpu_sc as plsc

_EMB = 64
_LANES = 16
_SEGS = _EMB // _LANES
_NC, _NS = 2, 16
_NW = _NC * _NS
_TILE = 128
_NTILES = 244
_EXTRA0 = _NW * _NTILES
_TAIL0 = 999936
_NBKT = 246
_BCAP = 16
_LCAP = 1024


def _phase1(center_words, context_words, in_t, out_t, tail_in, tail_out):
    B = center_words.shape[0]
    mesh = plsc.VectorSubcoreMesh(core_axis_name="c", subcore_axis_name="s")

    @functools.partial(
        pl.kernel,
        mesh=mesh,
        compiler_params=pltpu.CompilerParams(needs_layout_passes=False,
                                             use_tc_tiling_on_sc=True),
        out_type=(jax.ShapeDtypeStruct((B, _EMB), jnp.float32),
                  jax.ShapeDtypeStruct((B, _EMB), jnp.float32)),
        scratch_types=[
            pltpu.VMEM((B,), jnp.int32),
            pltpu.VMEM((5, _EMB, _TILE), jnp.float32),
            pltpu.VMEM((5, _EMB, _TILE), jnp.float32),
            pltpu.VMEM((_LCAP,), jnp.int32),
            pltpu.VMEM((_LCAP,), jnp.int32),
            pltpu.VMEM((_NBKT * _BCAP + 16,), jnp.int32),
            pltpu.VMEM((_NBKT * _BCAP + 16,), jnp.int32),
            pltpu.VMEM((_NBKT * _BCAP + 16,), jnp.int32),
            pltpu.VMEM((_NBKT * _BCAP + 16,), jnp.int32),
            pltpu.VMEM((256,), jnp.int32),
            pltpu.VMEM((256,), jnp.int32),
            pltpu.VMEM((32, _EMB), jnp.float32),
            pltpu.SemaphoreType.DMA,
            pltpu.SemaphoreType.DMA,
            pltpu.SemaphoreType.DMA,
            pltpu.SemaphoreType.DMA,
            pltpu.SemaphoreType.DMA,
            pltpu.SemaphoreType.DMA,
        ],
    )
    def p1(center_hbm, context_hbm, in_hbm, out_hbm, tin_hbm, tout_hbm,
           crows_hbm, xrows_hbm,
           idxbuf, wbin, wbout, mlidx, mlpos,
           cbcol, cbpos, xbcol, xbpos, ccnt, xcnt, stg,
           sem0, sem1, sem2, sem3, sem4, semo):
        wid = lax.axis_index("s") * _NC + lax.axis_index("c")
        lane = lax.iota(jnp.int32, _LANES)

        tile0 = _NTILES * wid
        lo = tile0 * _TILE
        hi = lo + _NTILES * _TILE
        has_extra = wid < 4
        elo = (_EXTRA0 + wid) * _TILE
        is_last = wid == (_NW - 1)
        lo2 = jnp.where(has_extra, elo, jnp.where(is_last, _TAIL0, -1))
        hi2 = jnp.where(has_extra, elo + _TILE,
                        jnp.where(is_last, 1 << 30, -1))

        sems = (sem0, sem1, sem2, sem3, sem4)

        def issue(w, slot):
            s = pl.ds((tile0 + w) * _TILE, _TILE)
            pltpu.async_copy(in_hbm.at[:, s], wbin.at[slot], sems[slot])
            pltpu.async_copy(out_hbm.at[:, s], wbout.at[slot], sems[slot])

        def drain(slot):
            pltpu.make_async_copy(in_hbm.at[:, pl.ds(0, _TILE)],
                                  wbin.at[slot], sems[slot]).wait()
            pltpu.make_async_copy(out_hbm.at[:, pl.ds(0, _TILE)],
                                  wbout.at[slot], sems[slot]).wait()

        for p in range(5):
            issue(p, p)

        def zero(j, z):
            ccnt[pl.ds(j * 16, 16)] = jnp.zeros((16,), jnp.int32)
            xcnt[pl.ds(j * 16, 16)] = jnp.zeros((16,), jnp.int32)
            return z

        lax.fori_loop(0, 16, zero, 0)

        def select(ibuf):
            def body(t, cnt):
                v = ibuf[pl.ds(t * 16, 16)]
                m = jnp.logical_or(
                    jnp.logical_and(v >= lo, v < hi),
                    jnp.logical_and(v >= lo2, v < hi2))
                plsc.store_compressed(mlidx.at[pl.ds(cnt, 16)], v, mask=m)
                plsc.store_compressed(mlpos.at[pl.ds(cnt, 16)],
                                      t * 16 + lane, mask=m)
                n = plsc.all_reduce_population_count(m)
                return cnt + n[0]

            return lax.fori_loop(0, B // 16, body, 0, unroll=4)

        lane0 = lane == 0

        def bucketize(cnt, bcol, bpos, bcnt):
            def body(j, z):
                idx = mlidx[pl.ds(j, 16)][0]
                pos = mlpos[pl.ds(j, 16)][0]
                t = jnp.where(idx >= _TAIL0, _NBKT - 1,
                              jnp.where(idx >= elo, _NBKT - 2,
                                        (idx >> 7) - tile0))
                col = jnp.where(idx >= _TAIL0, idx - _TAIL0,
                                jnp.bitwise_and(idx, _TILE - 1))
                tv = jax.lax.broadcast(t, (16,))
                k = plsc.load_gather(bcnt, [tv])[0]
                sv16 = jax.lax.broadcast(t * _BCAP + k, (16,))
                plsc.store_scatter(bcol, [sv16],
                                   jax.lax.broadcast(col, (16,)), mask=lane0)
                plsc.store_scatter(bpos, [sv16],
                                   jax.lax.broadcast(pos, (16,)), mask=lane0)
                plsc.store_scatter(bcnt, [tv],
                                   jax.lax.broadcast(k + 1, (16,)), mask=lane0)
                return z

            lax.fori_loop(0, cnt, body, 0)

        pltpu.sync_copy(center_hbm, idxbuf)
        ncm = select(idxbuf)
        bucketize(ncm, cbcol, cbpos, ccnt)
        pltpu.sync_copy(context_hbm, idxbuf)
        nxm = select(idxbuf)
        bucketize(nxm, xbcol, xbpos, xcnt)

        def extract(w, src, bcol, bpos, bcnt, rows_hbm, ne0):
            k = plsc.load_gather(bcnt, [jax.lax.broadcast(w, (16,))])[0]

            def ent(j, ne):
                col = bcol[pl.ds(w * _BCAP + j, 16)][0]
                pos = bpos[pl.ds(w * _BCAP + j, 16)][0]
                s = jnp.bitwise_and(ne, 31)

                @pl.when(ne >= 8)
                def _():
                    pltpu.make_async_copy(stg.at[0], rows_hbm.at[0],
                                          semo).wait()

                cv = jax.lax.broadcast(col, (16,))
                for seg in range(_SEGS):
                    g = plsc.load_gather(src, [seg * 16 + lane, cv])
                    stg[s, pl.ds(seg * 16, 16)] = g
                pltpu.async_copy(stg.at[s], rows_hbm.at[pos], semo)
                return ne + 1

            return lax.fori_loop(0, k, ent, ne0)

        def step(u, ne):
            for p in range(5):
                w = 5 * u + p
                drain(p)
                ne = extract(w, wbin.at[p], cbcol, cbpos, ccnt, crows_hbm, ne)
                ne = extract(w, wbout.at[p], xbcol, xbpos, xcnt, xrows_hbm, ne)

                @pl.when(w + 5 < _NTILES)
                def _():
                    issue(w + 5, p)

            return ne

        ne = lax.fori_loop(0, _NTILES // 5, step, 0)

        for p in range(4):
            w = (_NTILES // 5) * 5 + p
            drain(p)
            ne = extract(w, wbin.at[p], cbcol, cbpos, ccnt, crows_hbm, ne)
            ne = extract(w, wbout.at[p], xbcol, xbpos, xcnt, xrows_hbm, ne)

        def extra_win(n):
            s = pl.ds((_EXTRA0 + wid) * _TILE, _TILE)
            pltpu.async_copy(in_hbm.at[:, s], wbin.at[0], sems[0])
            pltpu.async_copy(out_hbm.at[:, s], wbout.at[0], sems[0])
            drain(0)
            n = extract(_NBKT - 2, wbin.at[0], cbcol, cbpos, ccnt,
                        crows_hbm, n)
            n = extract(_NBKT - 2, wbout.at[0], xbcol, xbpos, xcnt,
                        xrows_hbm, n)
            return n

        ne = lax.cond(has_extra, extra_win, lambda n: n, ne)

        @pl.when(is_last)
        def _():
            pltpu.sync_copy(tin_hbm, wbin.at[1])
            pltpu.sync_copy(tout_hbm, wbout.at[1])

        ne = extract(_NBKT - 1, wbin.at[1], cbcol, cbpos, ccnt,
                     crows_hbm, ne)
        ne = extract(_NBKT - 1, wbout.at[1], xbcol, xbpos, xcnt,
                     xrows_hbm, ne)

        def fin(j, z):
            pltpu.make_async_copy(stg.at[0], crows_hbm.at[0], semo).wait()
            return z

        lax.fori_loop(0, jnp.minimum(ne, 8), fin, 0)

    return p1


def _phase2(B):
    b_per_w = B // _NW
    CH = 128
    n_chunks = b_per_w // CH
    mesh = plsc.VectorSubcoreMesh(core_axis_name="c", subcore_axis_name="s")

    @functools.partial(
        pl.kernel,
        mesh=mesh,
        compiler_params=pltpu.CompilerParams(needs_layout_passes=False,
                                             use_tc_tiling_on_sc=True),
        out_type=jax.ShapeDtypeStruct((B,), jnp.float32),
        scratch_types=[
            pltpu.VMEM((2, CH, _EMB), jnp.float32),
            pltpu.VMEM((2, CH, _EMB), jnp.float32),
            pltpu.VMEM((b_per_w,), jnp.float32),
            pltpu.SemaphoreType.DMA,
            pltpu.SemaphoreType.DMA,
        ],
    )
    def p2(crows_hbm, xrows_hbm, scores_hbm, cbuf, xbuf, sv, sem0, sem1):
        wid = lax.axis_index("s") * _NC + lax.axis_index("c")
        base = wid * b_per_w
        sems = (sem0, sem1)
        lane = lax.iota(jnp.int32, _LANES)

        def issue(c, slot):
            s = pl.ds(base + c * CH, CH)
            pltpu.async_copy(crows_hbm.at[s], cbuf.at[slot], sems[slot])
            pltpu.async_copy(xrows_hbm.at[s], xbuf.at[slot], sems[slot])

        def drain(slot):
            pltpu.make_async_copy(crows_hbm.at[pl.ds(0, CH)],
                                  cbuf.at[slot], sems[slot]).wait()
            pltpu.make_async_copy(xrows_hbm.at[pl.ds(0, CH)],
                                  xbuf.at[slot], sems[slot]).wait()

        def compute(c, slot):
            for g in range(CH // _LANES):
                res = jnp.zeros((_LANES,), jnp.float32)
                for i in range(_LANES):
                    li = g * _LANES + i
                    acc = (cbuf[slot, li, pl.ds(0, _LANES)]
                           * xbuf[slot, li, pl.ds(0, _LANES)])
                    for s in range(1, _SEGS):
                        acc = acc + (
                            cbuf[slot, li, pl.ds(s * _LANES, _LANES)]
                            * xbuf[slot, li, pl.ds(s * _LANES, _LANES)])
                    res = jnp.where(lane == i, jnp.sum(acc), res)
                sv[pl.ds(c * CH + g * _LANES, _LANES)] = res

        issue(0, 0)
        issue(1, 1)

        def step(t, carry):
            drain(0)
            compute(2 * t, 0)

            @pl.when(t < n_chunks // 2 - 1)
            def _():
                issue(2 * t + 2, 0)

            drain(1)
            compute(2 * t + 1, 1)

            @pl.when(t < n_chunks // 2 - 1)
            def _():
                issue(2 * t + 3, 1)

            return carry

        lax.fori_loop(0, n_chunks // 2, step, 0)
        pltpu.sync_copy(sv, scores_hbm.at[pl.ds(base, b_per_w)])

    return p2


def kernel(center_words, context_words, in_emb, out_emb):
    B = center_words.shape[0]
    in_t = in_emb.T
    out_t = out_emb.T
    tail_in = jnp.pad(in_emb[_TAIL0:].T, ((0, 0), (0, 64)))
    tail_out = jnp.pad(out_emb[_TAIL0:].T, ((0, 0), (0, 64)))
    crows, xrows = _phase1(center_words, context_words, in_t, out_t,
                           tail_in, tail_out)(
        center_words, context_words, in_t, out_t, tail_in, tail_out)
    return _phase2(B)(crows, xrows)
